# baseline (speedup 1.0000x reference)
import jax
import jax.numpy as jnp
from jax import lax
from jax.experimental import pallas as pl
from jax.experimental.pallas import tpu as pltpu
from jax.experimental.pallas import tpu_sc as plsc

N = 10000
D = 128
E = 320000

NC = 2
NS = 16
NW = NC * NS
CH = 128
NCHUNK = 80
EPW = CH * NCHUNK
CHG = 32
NQ = 4
CHB = CHG * NQ
NBCH = EPW // CHB
GRPB = 8
NGRPB = NBCH // GRPB
E_PAD = EPW * NW
N_PAD = 10240
RPT = N_PAD // NS

_MESH = dict(core_axis_name="c", subcore_axis_name="s",
             num_cores=NC, num_subcores=NS)


def _sc_scatter_body(src_hbm, dst_hbm, y_hbm, zero_hbm, out_hbm,
                     src_g, dst_g, r0, r1, acc,
                     g0, g1, g2, g3, g4, g5, g6, g7):
    rows = (r0, r1)
    gsem = ((g0, g1, g2, g3), (g4, g5, g6, g7))
    cid = lax.axis_index("c")
    sid = lax.axis_index("s")
    wid = cid * NS + sid

    def gather_batch(j, b):
        for q in range(NQ):
            pltpu.async_copy(
                y_hbm.at[src_g.at[j * NQ + q]],
                rows[b].at[pl.ds(q * CHG, CHG)], gsem[b][q])

    def wait_batch(b):
        for q in range(NQ):
            pltpu.make_async_copy(y_hbm.at[pl.ds(0, CHG)],
                                  rows[b].at[pl.ds(q * CHG, CHG)],
                                  gsem[b][q]).wait()

    pltpu.sync_copy(zero_hbm, acc.at[pl.ds(sid * RPT, RPT)])
    plsc.subcore_barrier()

    def group(g, carry):
        pltpu.sync_copy(
            src_hbm.at[wid, pl.ds(g * GRPB * NQ, GRPB * NQ)], src_g)
        pltpu.sync_copy(dst_hbm.at[wid, pl.ds(g * GRPB, GRPB)], dst_g)
        gather_batch(0, 0)
        gather_batch(1, 1)

        def pair(p, c2):
            j = p * 2
            for b in range(2):
                wait_batch(b)
                pltpu.sync_copy(rows[b], acc.at[dst_g.at[j + b]], add=True)
                gather_batch(j + 2 + b, b)
            return c2

        lax.fori_loop(0, GRPB // 2 - 1, pair, 0)
        for b in range(2):
            wait_batch(b)
            pltpu.sync_copy(rows[b], acc.at[dst_g.at[GRPB - 2 + b]], add=True)
        return carry

    lax.fori_loop(0, NGRPB, group, 0)

    plsc.subcore_barrier()
    pltpu.sync_copy(acc.at[pl.ds(sid * RPT, RPT)],
                    out_hbm.at[cid, pl.ds(sid * RPT, RPT)])


def _sc_scatter(src4, dst2, y, zero_rows):
    return pl.kernel(
        _sc_scatter_body,
        out_type=jax.ShapeDtypeStruct((NC, N_PAD, D), jnp.float32),
        mesh=plsc.VectorSubcoreMesh(**_MESH),
        scratch_types=[
            pltpu.VMEM((GRPB * NQ, CHG), jnp.int32),
            pltpu.VMEM((GRPB, CHB), jnp.int32),
            pltpu.VMEM((CHB, D), jnp.float32),
            pltpu.VMEM((CHB, D), jnp.float32),
            pltpu.VMEM_SHARED((N_PAD, D), jnp.float32),
            pltpu.SemaphoreType.DMA,
            pltpu.SemaphoreType.DMA,
            pltpu.SemaphoreType.DMA,
            pltpu.SemaphoreType.DMA,
            pltpu.SemaphoreType.DMA,
            pltpu.SemaphoreType.DMA,
            pltpu.SemaphoreType.DMA,
            pltpu.SemaphoreType.DMA,
        ],
    )(src4, dst2, y, zero_rows)


NDEG = 10240
RPTD = NDEG // NS


def _sc_deg_body(dst_hbm, out_hbm, dst_all, hist, red, shared):
    cid = lax.axis_index("c")
    sid = lax.axis_index("s")
    wid = cid * NS + sid

    pltpu.sync_copy(dst_hbm.at[wid], dst_all)

    def zero16v(i, carry):
        hist[pl.ds(i * 16, 16)] = jnp.zeros((16,), jnp.float32)
        return carry

    lax.fori_loop(0, NDEG // 16, zero16v, 0)

    ones_v = jnp.ones((16,), jnp.float32)

    def chunk(ch, carry):
        for j in range(CH // 16):
            idx = dst_all[ch, pl.ds(j * 16, 16)]
            plsc.addupdate_scatter(hist, [idx], ones_v)
        return carry

    lax.fori_loop(0, NCHUNK, chunk, 0)

    pltpu.sync_copy(hist, shared.at[sid])
    plsc.subcore_barrier()
    pltpu.sync_copy(shared.at[:, pl.ds(sid * RPTD, RPTD)], red)

    def add_rows(r, carry):
        def addv(i, c2):
            red[0, pl.ds(i * 16, 16)] = (red[0, pl.ds(i * 16, 16)]
                                         + red[r, pl.ds(i * 16, 16)])
            return c2
        lax.fori_loop(0, RPTD // 16, addv, 0)
        return carry

    lax.fori_loop(1, NS, add_rows, 0)
    pltpu.sync_copy(red.at[0], out_hbm.at[cid, pl.ds(sid * RPTD, RPTD)])


def _sc_deg(dst3):
    return pl.kernel(
        _sc_deg_body,
        out_type=jax.ShapeDtypeStruct((NC, NDEG), jnp.float32),
        mesh=plsc.VectorSubcoreMesh(**_MESH),
        scratch_types=[
            pltpu.VMEM((NCHUNK, CH), jnp.int32),
            pltpu.VMEM((NDEG,), jnp.float32),
            pltpu.VMEM((NS, RPTD), jnp.float32),
            pltpu.VMEM_SHARED((NS, NDEG), jnp.float32),
        ],
        compiler_params=pltpu.CompilerParams(needs_layout_passes=False),
    )(dst3)


R = 2048
G = N_PAD // R


def _dis_col(degp_blk):
    deg_row = jnp.sum(degp_blk, axis=0, keepdims=True) + 1.0
    return lax.rsqrt(jnp.transpose(deg_row, (1, 0)))


def _tc_a1_body(x_ref, w_ref, degp_ref, y_ref):
    dis = _dis_col(degp_ref[...])
    xw = jnp.dot(x_ref[...], w_ref[...], preferred_element_type=jnp.float32)
    y_ref[...] = dis * xw


def _tc_a1(x, W1, degp):
    return pl.pallas_call(
        _tc_a1_body,
        grid=(G,),
        in_specs=[
            pl.BlockSpec((R, D), lambda i: (i, 0)),
            pl.BlockSpec((D, D), lambda i: (0, 0)),
            pl.BlockSpec((NC, R), lambda i: (0, i)),
        ],
        out_specs=pl.BlockSpec((R, D), lambda i: (i, 0)),
        out_shape=jax.ShapeDtypeStruct((N_PAD, D), jnp.float32),
    )(x, W1, degp)


def _tc_a2_body(aggp_ref, y_ref, degp_ref, b_ref, w_ref, out_ref):
    dis = _dis_col(degp_ref[...])
    s = aggp_ref[0] + aggp_ref[1] + y_ref[...]
    h = jnp.maximum(dis * s + b_ref[...], 0.0)
    out_ref[...] = dis * jnp.dot(
        h, w_ref[...], preferred_element_type=jnp.float32)


def _tc_a2(aggp, y1, degp, b1, W2):
    return pl.pallas_call(
        _tc_a2_body,
        grid=(G,),
        in_specs=[
            pl.BlockSpec((NC, R, D), lambda i: (0, i, 0)),
            pl.BlockSpec((R, D), lambda i: (i, 0)),
            pl.BlockSpec((NC, R), lambda i: (0, i)),
            pl.BlockSpec((1, D), lambda i: (0, 0)),
            pl.BlockSpec((D, D), lambda i: (0, 0)),
        ],
        out_specs=pl.BlockSpec((R, D), lambda i: (i, 0)),
        out_shape=jax.ShapeDtypeStruct((N_PAD, D), jnp.float32),
    )(aggp, y1, degp, b1, W2)


def _tc_a3_body(aggp_ref, y_ref, degp_ref, b_ref, out_ref):
    dis = _dis_col(degp_ref[...])
    z = dis * (aggp_ref[0] + aggp_ref[1] + y_ref[...]) + b_ref[...]
    m = jnp.max(z, axis=1, keepdims=True)
    s = jnp.sum(jnp.exp(z - m), axis=1, keepdims=True)
    out_ref[...] = z - m - jnp.log(s)


def _tc_a3(aggp, y2, degp, b2):
    return pl.pallas_call(
        _tc_a3_body,
        grid=(G,),
        in_specs=[
            pl.BlockSpec((NC, R, D), lambda i: (0, i, 0)),
            pl.BlockSpec((R, D), lambda i: (i, 0)),
            pl.BlockSpec((NC, R), lambda i: (0, i)),
            pl.BlockSpec((1, D), lambda i: (0, 0)),
        ],
        out_specs=pl.BlockSpec((R, D), lambda i: (i, 0)),
        out_shape=jax.ShapeDtypeStruct((N_PAD, D), jnp.float32),
    )(aggp, y2, degp, b2)


def kernel(x, edge_index, W1, b1, W2, b2):
    src = edge_index[0]
    dst = edge_index[1]
    pad = E_PAD - E
    ar = jnp.arange(pad, dtype=jnp.int32)
    src_p = jnp.concatenate([src, (ar * 13) % N])
    dst_p = jnp.concatenate([dst, N + (ar * 7) % (N_PAD - N)])
    src4 = src_p.reshape(NW, NBCH * NQ, CHG)
    dst2 = dst_p.reshape(NW, NBCH, CHB)
    dst3_deg = dst_p.reshape(NW, NCHUNK, CH)

    zero_rows = jnp.zeros((RPT, D), jnp.float32)
    x_pad = jnp.concatenate([x, jnp.zeros((N_PAD - N, D), jnp.float32)])
    degp = _sc_deg(dst3_deg)
    y1 = _tc_a1(x_pad, W1, degp)
    agg1 = _sc_scatter(src4, dst2, y1, zero_rows)
    y2 = _tc_a2(agg1, y1, degp, b1.reshape(1, D), W2)
    agg2 = _sc_scatter(src4, dst2, y2, zero_rows)
    return _tc_a3(agg2, y2, degp, b2.reshape(1, D))[:N]


# speedup vs baseline: 1.1877x; 1.1877x over previous
import jax
import jax.numpy as jnp
from jax import lax
from jax.experimental import pallas as pl
from jax.experimental.pallas import tpu as pltpu
from jax.experimental.pallas import tpu_sc as plsc

N = 10000
D = 128
E = 320000

NC = 2
NS = 16
NW = NC * NS
CH = 128
NCHUNK = 80
EPW = CH * NCHUNK
CHS = 64
NB = 4
NCHS = EPW // CHS
GRPS = 32
NGRPS = NCHS // GRPS
E_PAD = EPW * NW
N_PAD = 10240
RPT = N_PAD // NS

_MESH = dict(core_axis_name="c", subcore_axis_name="s",
             num_cores=NC, num_subcores=NS)


def _sc_scatter_body(src_hbm, dst_hbm, y_hbm, zero_hbm, out_hbm,
                     src_g, dst_g, r0, r1, r2, r3, acc, s0, s1, s2, s3):
    rows = (r0, r1, r2, r3)
    sems = (s0, s1, s2, s3)
    cid = lax.axis_index("c")
    sid = lax.axis_index("s")
    wid = cid * NS + sid

    pltpu.sync_copy(zero_hbm, acc.at[pl.ds(sid * RPT, RPT)])
    plsc.subcore_barrier()

    def group(g, carry):
        pltpu.sync_copy(src_hbm.at[wid, pl.ds(g * GRPS, GRPS)], src_g)
        pltpu.sync_copy(dst_hbm.at[wid, pl.ds(g * GRPS, GRPS)], dst_g)
        for b in range(NB):
            pltpu.async_copy(y_hbm.at[src_g.at[b]], rows[b], sems[b])

        def quad(q, c2):
            k = q * NB
            for b in range(NB):
                pltpu.make_async_copy(
                    y_hbm.at[pl.ds(0, CHS)], rows[b], sems[b]).wait()
                pltpu.sync_copy(rows[b], acc.at[dst_g.at[k + b]], add=True)
                pltpu.async_copy(
                    y_hbm.at[src_g.at[k + NB + b]], rows[b], sems[b])
            return c2

        lax.fori_loop(0, (GRPS - NB) // NB, quad, 0)
        for b in range(NB):
            pltpu.make_async_copy(
                y_hbm.at[pl.ds(0, CHS)], rows[b], sems[b]).wait()
            pltpu.sync_copy(rows[b], acc.at[dst_g.at[GRPS - NB + b]], add=True)
        return carry

    lax.fori_loop(0, NGRPS, group, 0)

    plsc.subcore_barrier()
    pltpu.sync_copy(acc.at[pl.ds(sid * RPT, RPT)],
                    out_hbm.at[cid, pl.ds(sid * RPT, RPT)])


def _sc_scatter(src3, dst3, y, zero_rows):
    return pl.kernel(
        _sc_scatter_body,
        out_type=jax.ShapeDtypeStruct((NC, N_PAD, D), jnp.float32),
        mesh=plsc.VectorSubcoreMesh(**_MESH),
        scratch_types=[
            pltpu.VMEM((GRPS, CHS), jnp.int32),
            pltpu.VMEM((GRPS, CHS), jnp.int32),
            pltpu.VMEM((CHS, D), jnp.float32),
            pltpu.VMEM((CHS, D), jnp.float32),
            pltpu.VMEM((CHS, D), jnp.float32),
            pltpu.VMEM((CHS, D), jnp.float32),
            pltpu.VMEM_SHARED((N_PAD, D), jnp.float32),
            pltpu.SemaphoreType.DMA,
            pltpu.SemaphoreType.DMA,
            pltpu.SemaphoreType.DMA,
            pltpu.SemaphoreType.DMA,
        ],
    )(src3, dst3, y, zero_rows)


NDEG = 10240
RPTD = NDEG // NS


def _sc_deg_body(dst_hbm, out_hbm, dst_all, hist, red, shared):
    cid = lax.axis_index("c")
    sid = lax.axis_index("s")
    wid = cid * NS + sid

    pltpu.sync_copy(dst_hbm.at[wid], dst_all)

    def zero16v(i, carry):
        hist[pl.ds(i * 16, 16)] = jnp.zeros((16,), jnp.float32)
        return carry

    lax.fori_loop(0, NDEG // 16, zero16v, 0)

    ones_v = jnp.ones((16,), jnp.float32)

    def chunk(ch, carry):
        for j in range(CH // 16):
            idx = dst_all[ch, pl.ds(j * 16, 16)]
            plsc.addupdate_scatter(hist, [idx], ones_v)
        return carry

    lax.fori_loop(0, NCHUNK, chunk, 0)

    pltpu.sync_copy(hist, shared.at[sid])
    plsc.subcore_barrier()
    pltpu.sync_copy(shared.at[:, pl.ds(sid * RPTD, RPTD)], red)

    def add_rows(r, carry):
        def addv(i, c2):
            red[0, pl.ds(i * 16, 16)] = (red[0, pl.ds(i * 16, 16)]
                                         + red[r, pl.ds(i * 16, 16)])
            return c2
        lax.fori_loop(0, RPTD // 16, addv, 0)
        return carry

    lax.fori_loop(1, NS, add_rows, 0)
    pltpu.sync_copy(red.at[0], out_hbm.at[cid, pl.ds(sid * RPTD, RPTD)])


def _sc_deg(dst3):
    return pl.kernel(
        _sc_deg_body,
        out_type=jax.ShapeDtypeStruct((NC, NDEG), jnp.float32),
        mesh=plsc.VectorSubcoreMesh(**_MESH),
        scratch_types=[
            pltpu.VMEM((NCHUNK, CH), jnp.int32),
            pltpu.VMEM((NDEG,), jnp.float32),
            pltpu.VMEM((NS, RPTD), jnp.float32),
            pltpu.VMEM_SHARED((NS, NDEG), jnp.float32),
        ],
        compiler_params=pltpu.CompilerParams(needs_layout_passes=False),
    )(dst3)


R = 2048
G = N_PAD // R


def _dis_col(degp_blk):
    deg_row = jnp.sum(degp_blk, axis=0, keepdims=True) + 1.0
    return lax.rsqrt(jnp.transpose(deg_row, (1, 0)))


def _tc_a1_body(x_ref, w_ref, degp_ref, y_ref):
    dis = _dis_col(degp_ref[...])
    xw = jnp.dot(x_ref[...], w_ref[...], preferred_element_type=jnp.float32)
    y_ref[...] = dis * xw


def _tc_a1(x, W1, degp):
    return pl.pallas_call(
        _tc_a1_body,
        grid=(G,),
        in_specs=[
            pl.BlockSpec((R, D), lambda i: (i, 0)),
            pl.BlockSpec((D, D), lambda i: (0, 0)),
            pl.BlockSpec((NC, R), lambda i: (0, i)),
        ],
        out_specs=pl.BlockSpec((R, D), lambda i: (i, 0)),
        out_shape=jax.ShapeDtypeStruct((N_PAD, D), jnp.float32),
    )(x, W1, degp)


def _tc_a2_body(aggp_ref, y_ref, degp_ref, b_ref, w_ref, out_ref):
    dis = _dis_col(degp_ref[...])
    s = aggp_ref[0] + aggp_ref[1] + y_ref[...]
    h = jnp.maximum(dis * s + b_ref[...], 0.0)
    out_ref[...] = dis * jnp.dot(
        h, w_ref[...], preferred_element_type=jnp.float32)


def _tc_a2(aggp, y1, degp, b1, W2):
    return pl.pallas_call(
        _tc_a2_body,
        grid=(G,),
        in_specs=[
            pl.BlockSpec((NC, R, D), lambda i: (0, i, 0)),
            pl.BlockSpec((R, D), lambda i: (i, 0)),
            pl.BlockSpec((NC, R), lambda i: (0, i)),
            pl.BlockSpec((1, D), lambda i: (0, 0)),
            pl.BlockSpec((D, D), lambda i: (0, 0)),
        ],
        out_specs=pl.BlockSpec((R, D), lambda i: (i, 0)),
        out_shape=jax.ShapeDtypeStruct((N_PAD, D), jnp.float32),
    )(aggp, y1, degp, b1, W2)


def _tc_a3_body(aggp_ref, y_ref, degp_ref, b_ref, out_ref):
    dis = _dis_col(degp_ref[...])
    z = dis * (aggp_ref[0] + aggp_ref[1] + y_ref[...]) + b_ref[...]
    m = jnp.max(z, axis=1, keepdims=True)
    s = jnp.sum(jnp.exp(z - m), axis=1, keepdims=True)
    out_ref[...] = z - m - jnp.log(s)


def _tc_a3(aggp, y2, degp, b2):
    return pl.pallas_call(
        _tc_a3_body,
        grid=(G,),
        in_specs=[
            pl.BlockSpec((NC, R, D), lambda i: (0, i, 0)),
            pl.BlockSpec((R, D), lambda i: (i, 0)),
            pl.BlockSpec((NC, R), lambda i: (0, i)),
            pl.BlockSpec((1, D), lambda i: (0, 0)),
        ],
        out_specs=pl.BlockSpec((R, D), lambda i: (i, 0)),
        out_shape=jax.ShapeDtypeStruct((N_PAD, D), jnp.float32),
    )(aggp, y2, degp, b2)


def kernel(x, edge_index, W1, b1, W2, b2):
    src = edge_index[0]
    dst = edge_index[1]
    pad = E_PAD - E
    ar = jnp.arange(pad, dtype=jnp.int32)
    src_p = jnp.concatenate([src, (ar * 13) % N])
    dst_p = jnp.concatenate([dst, N + (ar * 7) % (N_PAD - N)])
    src3 = src_p.reshape(NW, NCHS, CHS)
    dst3 = dst_p.reshape(NW, NCHS, CHS)
    dst3_deg = dst_p.reshape(NW, NCHUNK, CH)

    zero_rows = jnp.zeros((RPT, D), jnp.float32)
    x_pad = jnp.concatenate([x, jnp.zeros((N_PAD - N, D), jnp.float32)])
    degp = _sc_deg(dst3_deg)
    y1 = _tc_a1(x_pad, W1, degp)
    agg1 = _sc_scatter(src3, dst3, y1, zero_rows)
    y2 = _tc_a2(agg1, y1, degp, b1.reshape(1, D), W2)
    agg2 = _sc_scatter(src3, dst3, y2, zero_rows)
    return _tc_a3(agg2, y2, degp, b2.reshape(1, D))[:N]
